# baseline (speedup 1.0000x reference)
import functools

import jax
import jax.numpy as jnp
from jax import lax
from jax.experimental import pallas as pl
from jax.experimental.pallas import tpu as pltpu
from jax.experimental.pallas import tpu_sc as plsc

VOCAB = 1000000
EMBED = 64
OUT_DIM = 2
BATCH = 4096
HIST = 200

_NC = 2
_NS = 16
_NW = _NC * _NS
_BPW = BATCH // _NW
_HALF = HIST // 2
_NBUF = 4
_RPG = 8


def _sc_body(x2_hbm, emb_hbm, w_hbm, btile_hbm, out_hbm,
             idx_v, bufs, w_v, b_v, o_v, sems):
    wid = lax.axis_index("s") * _NC + lax.axis_index("c")

    pltpu.sync_copy(x2_hbm.at[pl.ds(wid * 2 * _BPW, 2 * _BPW)], idx_v)
    pltpu.sync_copy(w_hbm, w_v)
    pltpu.sync_copy(btile_hbm, b_v)

    ninf = jnp.full((32,), -jnp.inf, dtype=jnp.bfloat16)
    lane = lax.iota(jnp.int32, 16)

    _dnums = lax.GatherDimensionNumbers(
        offset_dims=(), collapsed_slice_dims=(0,), start_index_map=(0,))

    def lane_shuffle(p, perm):
        return lax.gather(p, perm.reshape(16, 1), _dnums, (1,),
                          mode=lax.GatherScatterMode.PROMISE_IN_BOUNDS)

    def lane_sum(p):
        for sh in (8, 4, 2, 1):
            p = p + lane_shuffle(p, jnp.bitwise_xor(lane, sh))
        return p

    def gather_row(b, k):
        r = 2 * b
        d0 = pltpu.async_copy(emb_hbm.at[idx_v.at[r]],
                              bufs.at[k, pl.ds(0, _HALF)], sems.at[k])
        d1 = pltpu.async_copy(emb_hbm.at[idx_v.at[r + 1]],
                              bufs.at[k, pl.ds(_HALF, _HALF)], sems.at[k])
        return d0, d1

    def pool_and_project(k, v, lane0):
        buf = bufs.at[k]

        def jbody(j, acc):
            lo, hi = acc
            return (jnp.maximum(lo, buf[j, 0:32]),
                    jnp.maximum(hi, buf[j, 32:64]))

        lo, hi = lax.fori_loop(0, HIST, jbody, (ninf, ninf), unroll=8)
        a0, a1 = plsc.unpack(lo, format=plsc.PackFormat.INTERLEAVED)
        a2, a3 = plsc.unpack(hi, format=plsc.PackFormat.INTERLEAVED)
        for o in range(OUT_DIM):
            p = (a0 * w_v[o, 0:16] + a1 * w_v[o, 16:32]
                 + a2 * w_v[o, 32:48] + a3 * w_v[o, 48:64])
            s = lane_sum(p)
            v = jnp.where(lane == (lane0 + o), s, v)
        return v

    def obody(i, _):
        b0 = _RPG * i
        descs = [None] * _NBUF
        for k in range(_NBUF):
            descs[k] = gather_row(b0 + k, k)
        v = jnp.zeros((16,), dtype=jnp.float32)
        for k in range(_RPG):
            slot = k % _NBUF
            d0, d1 = descs[slot]
            d0.wait()
            d1.wait()
            v = pool_and_project(slot, v, OUT_DIM * k)
            if k + _NBUF < _RPG:
                descs[slot] = gather_row(b0 + k + _NBUF, slot)
        o_v[i, :] = v
        return 0

    lax.fori_loop(0, _BPW // _RPG, obody, 0)

    for i in range(16):
        z = o_v[i, :] + b_v[i, :]
        o_v[i, :] = 1.0 / (1.0 + jnp.exp(-z))

    pltpu.sync_copy(o_v, out_hbm.at[wid])


@jax.jit
def _run(x2, emb_bf, w_perm, btile):
    mesh = plsc.VectorSubcoreMesh(core_axis_name="c", subcore_axis_name="s")
    f = functools.partial(
        pl.kernel,
        mesh=mesh,
        out_type=jax.ShapeDtypeStruct((_NW, 16, 16), jnp.float32),
        scratch_types=[
            pltpu.VMEM((2 * _BPW, _HALF), jnp.int32),
            pltpu.VMEM((_NBUF, HIST, EMBED), jnp.bfloat16),
            pltpu.VMEM((OUT_DIM, EMBED), jnp.float32),
            pltpu.VMEM((16, 16), jnp.float32),
            pltpu.VMEM((16, 16), jnp.float32),
            pltpu.SemaphoreType.DMA((_NBUF,)),
        ],
        compiler_params=pltpu.CompilerParams(
            use_tc_tiling_on_sc=False, needs_layout_passes=False),
    )(_sc_body)
    return f(x2, emb_bf, w_perm, btile)


def kernel(x, emb_weight, lin_w, lin_b):
    x2 = x.reshape(2 * BATCH, _HALF).astype(jnp.int32)
    emb_bf = emb_weight.astype(jnp.bfloat16)
    w_perm = jnp.concatenate(
        [lin_w[:, 0:32:2], lin_w[:, 1:32:2],
         lin_w[:, 32:64:2], lin_w[:, 33:64:2]], axis=1)
    btile = jnp.tile(lin_b, (16 * 16) // OUT_DIM).reshape(16, 16)
    out = _run(x2, emb_bf, w_perm, btile)
    return out.reshape(BATCH, OUT_DIM)


# speedup vs baseline: 1.2420x; 1.2420x over previous
import functools

import jax
import jax.numpy as jnp
from jax import lax
from jax.experimental import pallas as pl
from jax.experimental.pallas import tpu as pltpu
from jax.experimental.pallas import tpu_sc as plsc

VOCAB = 1000000
EMBED = 64
OUT_DIM = 2
BATCH = 4096
HIST = 200

_NC = 2
_NS = 16
_NW = _NC * _NS
_BPW = BATCH // _NW
_HALF = HIST // 2
_NBUF = 4
_RPG = 8


def _sc_body(x2_hbm, emb_hbm, w_hbm, btile_hbm, out_hbm,
             idx_v, bufs, w_v, b_v, o_v, sems):
    wid = lax.axis_index("s") * _NC + lax.axis_index("c")

    pltpu.sync_copy(x2_hbm.at[pl.ds(wid * 2 * _BPW, 2 * _BPW)], idx_v)
    pltpu.sync_copy(w_hbm, w_v)
    pltpu.sync_copy(btile_hbm, b_v)

    ninf = jnp.full((16,), -jnp.inf, dtype=jnp.float32)
    lane = lax.iota(jnp.int32, 16)

    _dnums = lax.GatherDimensionNumbers(
        offset_dims=(), collapsed_slice_dims=(0,), start_index_map=(0,))

    def lane_shuffle(p, perm):
        return lax.gather(p, perm.reshape(16, 1), _dnums, (1,),
                          mode=lax.GatherScatterMode.PROMISE_IN_BOUNDS)

    def lane_sum(p):
        for sh in (8, 4, 2, 1):
            p = p + lane_shuffle(p, jnp.bitwise_xor(lane, sh))
        return p

    def gather_row(b, k):
        r = 2 * b
        d0 = pltpu.async_copy(emb_hbm.at[idx_v.at[r]],
                              bufs.at[k, pl.ds(0, _HALF)], sems.at[k])
        d1 = pltpu.async_copy(emb_hbm.at[idx_v.at[r + 1]],
                              bufs.at[k, pl.ds(_HALF, _HALF)], sems.at[k])
        return d0, d1

    def pool_and_project(k, v, lane0):
        buf = bufs.at[k]

        def jbody(j, acc):
            a0, a1, a2, a3 = acc
            return (jnp.maximum(a0, buf[j, 0:16]),
                    jnp.maximum(a1, buf[j, 16:32]),
                    jnp.maximum(a2, buf[j, 32:48]),
                    jnp.maximum(a3, buf[j, 48:64]))

        a0, a1, a2, a3 = lax.fori_loop(0, HIST, jbody,
                                       (ninf, ninf, ninf, ninf), unroll=8)
        for o in range(OUT_DIM):
            p = (a0 * w_v[o, 0:16] + a1 * w_v[o, 16:32]
                 + a2 * w_v[o, 32:48] + a3 * w_v[o, 48:64])
            s = lane_sum(p)
            v = jnp.where(lane == (lane0 + o), s, v)
        return v

    def obody(i, _):
        b0 = _RPG * i
        descs = [None] * _NBUF
        for k in range(_NBUF):
            descs[k] = gather_row(b0 + k, k)
        v = jnp.zeros((16,), dtype=jnp.float32)
        for k in range(_RPG):
            slot = k % _NBUF
            d0, d1 = descs[slot]
            d0.wait()
            d1.wait()
            v = pool_and_project(slot, v, OUT_DIM * k)
            if k + _NBUF < _RPG:
                descs[slot] = gather_row(b0 + k + _NBUF, slot)
        o_v[i, :] = v
        return 0

    lax.fori_loop(0, _BPW // _RPG, obody, 0)

    for i in range(16):
        z = o_v[i, :] + b_v[i, :]
        o_v[i, :] = 1.0 / (1.0 + jnp.exp(-z))

    pltpu.sync_copy(o_v, out_hbm.at[wid])


@jax.jit
def _run(x2, emb_weight, lin_w, btile):
    mesh = plsc.VectorSubcoreMesh(core_axis_name="c", subcore_axis_name="s")
    f = functools.partial(
        pl.kernel,
        mesh=mesh,
        out_type=jax.ShapeDtypeStruct((_NW, 16, 16), jnp.float32),
        scratch_types=[
            pltpu.VMEM((2 * _BPW, _HALF), jnp.int32),
            pltpu.VMEM((_NBUF, HIST, EMBED), jnp.float32),
            pltpu.VMEM((OUT_DIM, EMBED), jnp.float32),
            pltpu.VMEM((16, 16), jnp.float32),
            pltpu.VMEM((16, 16), jnp.float32),
            pltpu.SemaphoreType.DMA((_NBUF,)),
        ],
        compiler_params=pltpu.CompilerParams(use_tc_tiling_on_sc=False),
    )(_sc_body)
    return f(x2, emb_weight, lin_w, btile)


def kernel(x, emb_weight, lin_w, lin_b):
    x2 = x.reshape(2 * BATCH, _HALF).astype(jnp.int32)
    btile = jnp.tile(lin_b, (16 * 16) // OUT_DIM).reshape(16, 16)
    out = _run(x2, emb_weight, lin_w, btile)
    return out.reshape(BATCH, OUT_DIM)
